# baseline (speedup 1.0000x reference)
import functools

import jax
import jax.numpy as jnp
from jax import lax
from jax.experimental import pallas as pl
from jax.experimental.pallas import tpu as pltpu
from jax.experimental.pallas import tpu_sc as plsc

N = 10000
E = 320000
IN_DIM = 128
HID = 64
HEADS1 = 8
OUT_DIM = 128
G = 16

NPAD = 10240
NW = 32
CHUNK = 128
CPT = 82
EPAD = NW * CPT * CHUNK
ROWS_PER_TILE = NPAD // 16

_NEG = -1e30


def _mm1_body(x_ref, w_ref, m_ref, ms_ref, h_ref, t_ref, ts_ref):
    h = pl.program_id(1)
    blk = jnp.dot(x_ref[...], w_ref[0], preferred_element_type=jnp.float32)
    h_ref[0] = blk
    t = jnp.dot(blk, m_ref[0], preferred_element_type=jnp.float32)
    ts = jnp.dot(blk, ms_ref[0], preferred_element_type=jnp.float32)

    @pl.when(h == 0)
    def _():
        t_ref[...] = t
        ts_ref[...] = ts

    @pl.when(h != 0)
    def _():
        t_ref[...] += t
        ts_ref[...] += ts


def _mm1(xpad, w1r, m1, m1s):
    nblk = NPAD // 640
    return pl.pallas_call(
        _mm1_body,
        grid=(nblk, HEADS1),
        in_specs=[
            pl.BlockSpec((640, IN_DIM), lambda j, h: (j, 0)),
            pl.BlockSpec((1, IN_DIM, HID), lambda j, h: (h, 0, 0)),
            pl.BlockSpec((1, HID, 16), lambda j, h: (h, 0, 0)),
            pl.BlockSpec((1, HID, 16), lambda j, h: (h, 0, 0)),
        ],
        out_specs=[
            pl.BlockSpec((1, 640, HID), lambda j, h: (h, j, 0)),
            pl.BlockSpec((640, 16), lambda j, h: (j, 0)),
            pl.BlockSpec((640, 16), lambda j, h: (j, 0)),
        ],
        out_shape=[
            jax.ShapeDtypeStruct((HEADS1, NPAD, HID), jnp.float32),
            jax.ShapeDtypeStruct((NPAD, 16), jnp.float32),
            jax.ShapeDtypeStruct((NPAD, 16), jnp.float32),
        ],
    )(xpad, w1r, m1, m1s)


_SC_PARAMS = pltpu.CompilerParams(use_tc_tiling_on_sc=False)


def _attn_sc(tmain, tswap, src, dst):
    mesh = plsc.VectorSubcoreMesh(core_axis_name="c", subcore_axis_name="s")

    @functools.partial(
        pl.kernel,
        mesh=mesh,
        compiler_params=_SC_PARAMS,
        out_type=[
            jax.ShapeDtypeStruct((EPAD, 16), jnp.float32),
            jax.ShapeDtypeStruct((2 * NPAD, 16), jnp.float32),
        ],
        scratch_types=[
            pltpu.VMEM((CHUNK,), jnp.int32),
            pltpu.VMEM((CHUNK,), jnp.int32),
            pltpu.VMEM((CHUNK, 16), jnp.float32),
            pltpu.VMEM((CHUNK, 16), jnp.float32),
            pltpu.VMEM((CHUNK, 16), jnp.float32),
            pltpu.VMEM((ROWS_PER_TILE, 16), jnp.float32),
            pltpu.VMEM_SHARED((NPAD, 16), jnp.float32),
        ],
    )
    def k(tm_hbm, tsw_hbm, src_hbm, dst_hbm, ex_hbm, dpart_hbm,
          src_v, dst_v, ts_v, td_v, ex_v, zbuf, shared):
        c = lax.axis_index("c")
        s = lax.axis_index("s")
        wid = s * 2 + c
        lmask = lax.iota(jnp.int32, 16) < 8

        @pl.loop(0, ROWS_PER_TILE)
        def _(i):
            zbuf[i, :] = jnp.zeros((16,), jnp.float32)

        pltpu.sync_copy(zbuf, shared.at[pl.ds(s * ROWS_PER_TILE, ROWS_PER_TILE)])
        plsc.subcore_barrier()

        @pl.loop(0, CPT)
        def _(t):
            base = (wid * CPT + t) * CHUNK
            pltpu.sync_copy(src_hbm.at[pl.ds(base, CHUNK)], src_v)
            pltpu.sync_copy(dst_hbm.at[pl.ds(base, CHUNK)], dst_v)
            pltpu.sync_copy(tm_hbm.at[src_v], ts_v)
            pltpu.sync_copy(tsw_hbm.at[dst_v], td_v)

            @pl.loop(0, CHUNK)
            def _(i):
                su = ts_v[i, :] + td_v[i, :]
                lr = jnp.maximum(su, 0.2 * su)
                exf = jnp.exp(lr)
                ex_v[i, :] = jnp.where(lmask, exf, 0.0)

            pltpu.sync_copy(ex_v, ex_hbm.at[pl.ds(base, CHUNK)])
            pltpu.sync_copy(ex_v, shared.at[dst_v], add=True)

        plsc.subcore_barrier()
        off = c * NPAD + s * ROWS_PER_TILE
        pltpu.sync_copy(shared.at[pl.ds(s * ROWS_PER_TILE, ROWS_PER_TILE)],
                        dpart_hbm.at[pl.ds(off, ROWS_PER_TILE)])

    return k(tmain, tswap, src, dst)


def _msg_sc(table, ex, src2d, dst3d, nheads):
    mesh = plsc.VectorSubcoreMesh(core_axis_name="c", subcore_axis_name="s")

    @functools.partial(
        pl.kernel,
        mesh=mesh,
        compiler_params=_SC_PARAMS,
        out_type=jax.ShapeDtypeStruct((2 * nheads * NPAD, HID), jnp.float32),
        scratch_types=[
            pltpu.VMEM((CPT * CHUNK,), jnp.int32),
            pltpu.VMEM((CPT, CHUNK), jnp.int32),
            pltpu.VMEM((CHUNK,), jnp.int32),
            pltpu.VMEM((CHUNK,), jnp.int32),
            pltpu.VMEM((CHUNK, HID), jnp.float32),
            pltpu.VMEM((CHUNK, HID), jnp.float32),
            pltpu.VMEM((CHUNK, HID), jnp.float32),
            pltpu.VMEM((CHUNK, HID), jnp.float32),
            pltpu.VMEM((CHUNK, 16), jnp.float32),
            pltpu.VMEM((CHUNK, 16), jnp.float32),
            pltpu.VMEM_SHARED((NPAD, HID), jnp.float32),
            pltpu.SemaphoreType.DMA,
            pltpu.SemaphoreType.DMA,
            pltpu.SemaphoreType.DMA,
            pltpu.SemaphoreType.DMA,
            pltpu.SemaphoreType.DMA,
            pltpu.SemaphoreType.DMA,
        ],
    )
    def k(tab_hbm, ex_hbm, src_hbm, dst_hbm, opart_hbm,
          src_all, dst2d, idx0, idx1, ri0, ri1, ro0, ro1, exb0, exb1,
          shared,
          sg0, sg1, se0, se1, ss0, ss1):
        c = lax.axis_index("c")
        s = lax.axis_index("s")
        wid = s * 2 + c
        idx = (idx0, idx1)
        ri = (ri0, ri1)
        ro = (ro0, ro1)
        exb = (exb0, exb1)
        sg = (sg0, sg1)
        se = (se0, se1)
        ss = (ss0, ss1)

        pltpu.sync_copy(src_hbm.at[wid], src_all)
        pltpu.sync_copy(dst_hbm.at[wid], dst2d)

        def issue(t, b, h):
            if h == 0:
                gidx = src_all.at[pl.ds(t * CHUNK, CHUNK)]
            else:
                for i in range(0, CHUNK, 16):
                    idx[b][pl.ds(i, 16)] = (
                        src_all[pl.ds(t * CHUNK + i, 16)] + h * NPAD)
                gidx = idx[b]
            pltpu.async_copy(tab_hbm.at[gidx], ri[b], sg[b])
            ebase = (wid * CPT + t) * CHUNK
            pltpu.async_copy(ex_hbm.at[pl.ds(ebase, CHUNK)], exb[b], se[b])

        def step(t, b, h):
            pltpu.make_async_copy(tab_hbm.at[idx[b]], ri[b], sg[b]).wait()
            pltpu.make_async_copy(
                ex_hbm.at[pl.ds(0, CHUNK)], exb[b], se[b]).wait()

            @pl.when(t >= 2)
            def _():
                pltpu.make_async_copy(
                    ro[b], shared.at[dst2d.at[0]], ss[b]).wait()

            @pl.loop(0, CHUNK, step=8)
            def _(i0):
                for u in range(8):
                    i = i0 + u
                    a = exb[b][i, :][h]
                    for j in range(HID // 16):
                        sl = pl.ds(j * 16, 16)
                        ro[b][i, sl] = ri[b][i, sl] * a

            pltpu.async_copy(ro[b], shared.at[dst2d.at[t]], ss[b], add=True)

            @pl.when(t + 2 < CPT)
            def _():
                issue(t + 2, b, h)

        for h in range(nheads):
            @pl.loop(0, CHUNK)
            def _(i):
                for j in range(HID // 16):
                    ro0[i, pl.ds(j * 16, 16)] = jnp.zeros((16,), jnp.float32)

            for kk in range(ROWS_PER_TILE // CHUNK):
                pltpu.sync_copy(
                    ro0,
                    shared.at[pl.ds(s * ROWS_PER_TILE + kk * CHUNK, CHUNK)])
            plsc.subcore_barrier()

            issue(0, 0, h)
            issue(1, 1, h)

            @pl.loop(0, CPT, step=2)
            def _(t):
                step(t, 0, h)
                step(t + 1, 1, h)

            for b in range(2):
                pltpu.make_async_copy(
                    ro[b], shared.at[dst2d.at[0]], ss[b]).wait()

            plsc.subcore_barrier()
            off = (c * nheads + h) * NPAD + s * ROWS_PER_TILE
            pltpu.sync_copy(shared.at[pl.ds(s * ROWS_PER_TILE, ROWS_PER_TILE)],
                            opart_hbm.at[pl.ds(off, ROWS_PER_TILE)])

    return k(table, ex, src2d, dst3d)


def _act1_body(op_ref, dp_ref, b_ref, out_ref):
    dsum = dp_ref[0] + dp_ref[1]
    dinv = 1.0 / (dsum + 1e-16)
    acc = op_ref[0] + op_ref[1]
    parts = [acc[h] * dinv[:, h:h + 1] for h in range(HEADS1)]
    cat = jnp.concatenate(parts, axis=1)
    out_ref[...] = jnp.maximum(cat + b_ref[...], 0.0)


def _act1(opart, dpart, b1):
    blk = 1280
    nblk = NPAD // blk
    return pl.pallas_call(
        _act1_body,
        grid=(nblk,),
        in_specs=[
            pl.BlockSpec((2, HEADS1, blk, HID), lambda j: (0, 0, j, 0)),
            pl.BlockSpec((2, blk, 16), lambda j: (0, j, 0)),
            pl.BlockSpec((1, HEADS1 * HID), lambda j: (0, 0)),
        ],
        out_specs=pl.BlockSpec((blk, HEADS1 * HID), lambda j: (j, 0)),
        out_shape=jax.ShapeDtypeStruct((NPAD, HEADS1 * HID), jnp.float32),
    )(opart, dpart, b1)


def _mm2_body(x_ref, w_ref, m_ref, ms_ref, h_ref, t_ref, ts_ref):
    blk = jnp.dot(x_ref[...], w_ref[...], preferred_element_type=jnp.float32)
    h_ref[...] = blk
    t_ref[...] = jnp.dot(blk, m_ref[...], preferred_element_type=jnp.float32)
    ts_ref[...] = jnp.dot(blk, ms_ref[...], preferred_element_type=jnp.float32)


def _mm2(h1act, w2, m2, m2s):
    blk = 1280
    nblk = NPAD // blk
    return pl.pallas_call(
        _mm2_body,
        grid=(nblk,),
        in_specs=[
            pl.BlockSpec((blk, HEADS1 * HID), lambda j: (j, 0)),
            pl.BlockSpec((HEADS1 * HID, HID), lambda j: (0, 0)),
            pl.BlockSpec((HID, 16), lambda j: (0, 0)),
            pl.BlockSpec((HID, 16), lambda j: (0, 0)),
        ],
        out_specs=[
            pl.BlockSpec((blk, HID), lambda j: (j, 0)),
            pl.BlockSpec((blk, 16), lambda j: (j, 0)),
            pl.BlockSpec((blk, 16), lambda j: (j, 0)),
        ],
        out_shape=[
            jax.ShapeDtypeStruct((NPAD, HID), jnp.float32),
            jax.ShapeDtypeStruct((NPAD, 16), jnp.float32),
            jax.ShapeDtypeStruct((NPAD, 16), jnp.float32),
        ],
    )(h1act, w2, m2, m2s)


def _final_body(op_ref, dp_ref, b_ref, bt_ref, wfc_ref, bfc_ref, out_ref):
    d = dp_ref[0] + dp_ref[1]
    dinv = 1.0 / (d[:, 0:1] + 1e-16)
    acc = op_ref[0] + op_ref[1]
    h2act = jnp.maximum(acc * dinv + b_ref[...], 0.0)
    bt = bt_ref[...]
    gids = lax.broadcasted_iota(jnp.int32, (G, NPAD), 0)
    onehot = (bt == gids).astype(jnp.float32)
    pooled = jnp.dot(onehot, h2act, preferred_element_type=jnp.float32)
    out_ref[...] = jnp.dot(pooled, wfc_ref[...],
                           preferred_element_type=jnp.float32) + bfc_ref[...]


def _final(opart2, dpart2, b2, batch2d, wfc, bfc):
    return pl.pallas_call(
        _final_body,
        out_shape=jax.ShapeDtypeStruct((G, OUT_DIM), jnp.float32),
    )(opart2.reshape(2, NPAD, HID), dpart2.reshape(2, NPAD, 16),
      b2, batch2d, wfc, bfc)


def kernel(x, edge_index, batch, W1, att_src1, att_dst1, b1,
           W2, att_src2, att_dst2, b2, Wfc, bfc):
    f32 = jnp.float32

    xpad = jnp.pad(x, ((0, NPAD - N), (0, 0)))
    w1r = W1.reshape(IN_DIM, HEADS1, HID).transpose(1, 0, 2)

    eye8 = jnp.eye(HEADS1, dtype=f32)
    as1 = att_src1[0]
    ad1 = att_dst1[0]
    m1 = jnp.concatenate(
        [as1[:, :, None] * eye8[:, None, :],
         ad1[:, :, None] * eye8[:, None, :]], axis=2)
    m1s = jnp.concatenate([m1[:, :, 8:], m1[:, :, :8]], axis=2)

    m2 = jnp.zeros((HID, 16), f32)
    m2 = m2.at[:, 0].set(att_src2[0, 0]).at[:, 8].set(att_dst2[0, 0])
    m2s = jnp.concatenate([m2[:, 8:], m2[:, :8]], axis=1)

    loop = jnp.arange(N, dtype=jnp.int32)
    npadfill = jnp.full((EPAD - E - N,), N, jnp.int32)
    src = jnp.concatenate([edge_index[0], loop, npadfill])
    dst = jnp.concatenate([edge_index[1], loop, npadfill])
    src2d = src.reshape(NW, CPT * CHUNK)
    dst3d = dst.reshape(NW, CPT, CHUNK)

    rowid = jnp.arange(NPAD, dtype=jnp.int32)[:, None]
    batch2d = jnp.concatenate(
        [batch, jnp.full((NPAD - N,), G, jnp.int32)])[None, :]

    h1T, t1, t1s = _mm1(xpad, w1r, m1, m1s)
    t1 = jnp.where(rowid < N, t1, _NEG)
    t1s = jnp.where(rowid < N, t1s, _NEG)
    ex1, dpart1 = _attn_sc(t1, t1s, src, dst)
    opart1 = _msg_sc(h1T.reshape(HEADS1 * NPAD, HID), ex1, src2d, dst3d, HEADS1)
    h1act = _act1(opart1.reshape(2, HEADS1, NPAD, HID),
                  dpart1.reshape(2, NPAD, 16), b1[None, :])

    h2, t2, t2s = _mm2(h1act, W2, m2, m2s)
    t2 = jnp.where(rowid < N, t2, _NEG)
    t2s = jnp.where(rowid < N, t2s, _NEG)
    ex2, dpart2 = _attn_sc(t2, t2s, src, dst)
    opart2 = _msg_sc(h2, ex2, src2d, dst3d, 1)

    return _final(opart2, dpart2, b2[None, :], batch2d, Wfc, bfc[None, :])


# speedup vs baseline: 1.0032x; 1.0032x over previous
import functools

import jax
import jax.numpy as jnp
from jax import lax
from jax.experimental import pallas as pl
from jax.experimental.pallas import tpu as pltpu
from jax.experimental.pallas import tpu_sc as plsc

N = 10000
E = 320000
IN_DIM = 128
HID = 64
HEADS1 = 8
OUT_DIM = 128
G = 16

NPAD = 10240
NW = 32
CHUNK = 128
CPT = 82
EPAD = NW * CPT * CHUNK
ROWS_PER_TILE = NPAD // 16

_NEG = -1e30


def _mm1_body(x_ref, w_ref, m_ref, ms_ref, h_ref, t_ref, ts_ref):
    h = pl.program_id(1)
    blk = jnp.dot(x_ref[...], w_ref[0], preferred_element_type=jnp.float32)
    h_ref[0] = blk.astype(jnp.bfloat16)
    t = jnp.dot(blk, m_ref[0], preferred_element_type=jnp.float32)
    ts = jnp.dot(blk, ms_ref[0], preferred_element_type=jnp.float32)

    @pl.when(h == 0)
    def _():
        t_ref[...] = t
        ts_ref[...] = ts

    @pl.when(h != 0)
    def _():
        t_ref[...] += t
        ts_ref[...] += ts


def _mm1(xpad, w1r, m1, m1s):
    nblk = NPAD // 640
    return pl.pallas_call(
        _mm1_body,
        grid=(nblk, HEADS1),
        in_specs=[
            pl.BlockSpec((640, IN_DIM), lambda j, h: (j, 0)),
            pl.BlockSpec((1, IN_DIM, HID), lambda j, h: (h, 0, 0)),
            pl.BlockSpec((1, HID, 16), lambda j, h: (h, 0, 0)),
            pl.BlockSpec((1, HID, 16), lambda j, h: (h, 0, 0)),
        ],
        out_specs=[
            pl.BlockSpec((1, 640, HID), lambda j, h: (h, j, 0)),
            pl.BlockSpec((640, 16), lambda j, h: (j, 0)),
            pl.BlockSpec((640, 16), lambda j, h: (j, 0)),
        ],
        out_shape=[
            jax.ShapeDtypeStruct((HEADS1, NPAD, HID), jnp.bfloat16),
            jax.ShapeDtypeStruct((NPAD, 16), jnp.float32),
            jax.ShapeDtypeStruct((NPAD, 16), jnp.float32),
        ],
    )(xpad, w1r, m1, m1s)


_SC_PARAMS = pltpu.CompilerParams(use_tc_tiling_on_sc=False,
                                  needs_layout_passes=False)


def _attn_sc(tmain, tswap, src, dst):
    mesh = plsc.VectorSubcoreMesh(core_axis_name="c", subcore_axis_name="s")

    @functools.partial(
        pl.kernel,
        mesh=mesh,
        compiler_params=_SC_PARAMS,
        out_type=[
            jax.ShapeDtypeStruct((16, EPAD), jnp.float32),
            jax.ShapeDtypeStruct((2 * NPAD, 16), jnp.float32),
        ],
        scratch_types=[
            pltpu.VMEM((CHUNK,), jnp.int32),
            pltpu.VMEM((CHUNK,), jnp.int32),
            pltpu.VMEM((CHUNK, 16), jnp.float32),
            pltpu.VMEM((CHUNK, 16), jnp.float32),
            pltpu.VMEM((CHUNK, 16), jnp.float32),
            pltpu.VMEM((16, CHUNK), jnp.float32),
            pltpu.VMEM((ROWS_PER_TILE, 16), jnp.float32),
            pltpu.VMEM_SHARED((NPAD, 16), jnp.float32),
        ],
    )
    def k(tm_hbm, tsw_hbm, src_hbm, dst_hbm, ex_hbm, dpart_hbm,
          src_v, dst_v, ts_v, td_v, ex_v, ext_v, zbuf, shared):
        c = lax.axis_index("c")
        s = lax.axis_index("s")
        wid = s * 2 + c
        lane = lax.iota(jnp.int32, 16)
        lmask = lane < 8

        @pl.loop(0, ROWS_PER_TILE)
        def _(i):
            zbuf[i, :] = jnp.zeros((16,), jnp.float32)

        pltpu.sync_copy(zbuf, shared.at[pl.ds(s * ROWS_PER_TILE, ROWS_PER_TILE)])
        plsc.subcore_barrier()

        @pl.loop(0, CPT)
        def _(t):
            base = (wid * CPT + t) * CHUNK
            pltpu.sync_copy(src_hbm.at[pl.ds(base, CHUNK)], src_v)
            pltpu.sync_copy(dst_hbm.at[pl.ds(base, CHUNK)], dst_v)
            pltpu.sync_copy(tm_hbm.at[src_v], ts_v)
            pltpu.sync_copy(tsw_hbm.at[dst_v], td_v)

            @pl.loop(0, CHUNK)
            def _(i):
                su = ts_v[i, :] + td_v[i, :]
                lr = jnp.maximum(su, 0.2 * su)
                exf = jnp.exp(lr)
                exm = jnp.where(lmask, exf, 0.0)
                ex_v[i, :] = exm
                plsc.store_scatter(ext_v, [lane, jnp.full((16,), i, jnp.int32)],
                                   exm)

            pltpu.sync_copy(ext_v, ex_hbm.at[:, pl.ds(base, CHUNK)])
            pltpu.sync_copy(ex_v, shared.at[dst_v], add=True)

        plsc.subcore_barrier()
        off = c * NPAD + s * ROWS_PER_TILE
        pltpu.sync_copy(shared.at[pl.ds(s * ROWS_PER_TILE, ROWS_PER_TILE)],
                        dpart_hbm.at[pl.ds(off, ROWS_PER_TILE)])

    return k(tmain, tswap, src, dst)


def _msg_sc(table, ex, src2d, dst3d, nheads):
    mesh = plsc.VectorSubcoreMesh(core_axis_name="c", subcore_axis_name="s")

    @functools.partial(
        pl.kernel,
        mesh=mesh,
        compiler_params=_SC_PARAMS,
        out_type=jax.ShapeDtypeStruct((2 * nheads * NPAD, HID), jnp.float32),
        scratch_types=[
            pltpu.VMEM((CPT * CHUNK,), jnp.int32),
            pltpu.VMEM((CPT, CHUNK), jnp.int32),
            pltpu.VMEM((CHUNK,), jnp.int32),
            pltpu.VMEM((CHUNK,), jnp.int32),
            pltpu.VMEM((CHUNK, HID), jnp.bfloat16),
            pltpu.VMEM((CHUNK, HID), jnp.bfloat16),
            pltpu.VMEM((CHUNK, HID), jnp.float32),
            pltpu.VMEM((CHUNK, HID), jnp.float32),
            pltpu.VMEM((CHUNK,), jnp.float32),
            pltpu.VMEM((CHUNK,), jnp.float32),
            pltpu.VMEM_SHARED((NPAD, HID), jnp.float32),
            pltpu.SemaphoreType.DMA,
            pltpu.SemaphoreType.DMA,
            pltpu.SemaphoreType.DMA,
            pltpu.SemaphoreType.DMA,
            pltpu.SemaphoreType.DMA,
            pltpu.SemaphoreType.DMA,
        ],
    )
    def k(tab_hbm, ex_hbm, src_hbm, dst_hbm, opart_hbm,
          src_all, dst2d, idx0, idx1, ri0, ri1, ro0, ro1, exb0, exb1,
          shared,
          sg0, sg1, se0, se1, ss0, ss1):
        c = lax.axis_index("c")
        s = lax.axis_index("s")
        wid = s * 2 + c
        idx = (idx0, idx1)
        ri = (ri0, ri1)
        ro = (ro0, ro1)
        exb = (exb0, exb1)
        sg = (sg0, sg1)
        se = (se0, se1)
        ss = (ss0, ss1)

        pltpu.sync_copy(src_hbm.at[wid], src_all)
        pltpu.sync_copy(dst_hbm.at[wid], dst2d)

        def issue(t, b, h):
            if h == 0:
                gidx = src_all.at[pl.ds(t * CHUNK, CHUNK)]
            else:
                for i in range(0, CHUNK, 16):
                    idx[b][pl.ds(i, 16)] = (
                        src_all[pl.ds(t * CHUNK + i, 16)] + h * NPAD)
                gidx = idx[b]
            pltpu.async_copy(tab_hbm.at[gidx], ri[b], sg[b])
            ebase = (wid * CPT + t) * CHUNK
            pltpu.async_copy(ex_hbm.at[h, pl.ds(ebase, CHUNK)], exb[b], se[b])

        def step(t, b, h):
            pltpu.make_async_copy(tab_hbm.at[idx[b]], ri[b], sg[b]).wait()
            pltpu.make_async_copy(
                ex_hbm.at[0, pl.ds(0, CHUNK)], exb[b], se[b]).wait()

            @pl.when(t >= 2)
            def _():
                pltpu.make_async_copy(
                    ro[b], shared.at[dst2d.at[0]], ss[b]).wait()

            @pl.loop(0, CHUNK, step=16)
            def _(i0):
                av = exb[b][pl.ds(i0, 16)]
                for u in range(16):
                    i = i0 + u
                    a = av[u]
                    for j in range(HID // 32):
                        lo, hi = plsc.unpack(
                            ri[b][i, pl.ds(j * 32, 32)],
                            format=plsc.PackFormat.INTERLEAVED)
                        ro[b][i, pl.ds(j * 32, 16)] = lo * a
                        ro[b][i, pl.ds(j * 32 + 16, 16)] = hi * a

            pltpu.async_copy(ro[b], shared.at[dst2d.at[t]], ss[b], add=True)

            @pl.when(t + 2 < CPT)
            def _():
                issue(t + 2, b, h)

        for h in range(nheads):
            @pl.loop(0, CHUNK)
            def _(i):
                for j in range(HID // 16):
                    ro0[i, pl.ds(j * 16, 16)] = jnp.zeros((16,), jnp.float32)

            for kk in range(ROWS_PER_TILE // CHUNK):
                pltpu.sync_copy(
                    ro0,
                    shared.at[pl.ds(s * ROWS_PER_TILE + kk * CHUNK, CHUNK)])
            plsc.subcore_barrier()

            issue(0, 0, h)
            issue(1, 1, h)

            @pl.loop(0, CPT, step=2)
            def _(t):
                step(t, 0, h)
                step(t + 1, 1, h)

            for b in range(2):
                pltpu.make_async_copy(
                    ro[b], shared.at[dst2d.at[0]], ss[b]).wait()

            plsc.subcore_barrier()
            off = (c * nheads + h) * NPAD + s * ROWS_PER_TILE
            pltpu.sync_copy(shared.at[pl.ds(s * ROWS_PER_TILE, ROWS_PER_TILE)],
                            opart_hbm.at[pl.ds(off, ROWS_PER_TILE)])

    return k(table, ex, src2d, dst3d)


def _act1_body(op_ref, dp_ref, b_ref, out_ref):
    dsum = dp_ref[0] + dp_ref[1]
    dinv = 1.0 / (dsum + 1e-16)
    acc = op_ref[0] + op_ref[1]
    parts = [acc[h] * dinv[:, h:h + 1] for h in range(HEADS1)]
    cat = jnp.concatenate(parts, axis=1)
    out_ref[...] = jnp.maximum(cat + b_ref[...], 0.0)


def _act1(opart, dpart, b1):
    blk = 1280
    nblk = NPAD // blk
    return pl.pallas_call(
        _act1_body,
        grid=(nblk,),
        in_specs=[
            pl.BlockSpec((2, HEADS1, blk, HID), lambda j: (0, 0, j, 0)),
            pl.BlockSpec((2, blk, 16), lambda j: (0, j, 0)),
            pl.BlockSpec((1, HEADS1 * HID), lambda j: (0, 0)),
        ],
        out_specs=pl.BlockSpec((blk, HEADS1 * HID), lambda j: (j, 0)),
        out_shape=jax.ShapeDtypeStruct((NPAD, HEADS1 * HID), jnp.float32),
    )(opart, dpart, b1)


def _mm2_body(x_ref, w_ref, m_ref, ms_ref, h_ref, t_ref, ts_ref):
    blk = jnp.dot(x_ref[...], w_ref[...], preferred_element_type=jnp.float32)
    h_ref[...] = blk.astype(jnp.bfloat16)
    t_ref[...] = jnp.dot(blk, m_ref[...], preferred_element_type=jnp.float32)
    ts_ref[...] = jnp.dot(blk, ms_ref[...], preferred_element_type=jnp.float32)


def _mm2(h1act, w2, m2, m2s):
    blk = 1280
    nblk = NPAD // blk
    return pl.pallas_call(
        _mm2_body,
        grid=(nblk,),
        in_specs=[
            pl.BlockSpec((blk, HEADS1 * HID), lambda j: (j, 0)),
            pl.BlockSpec((HEADS1 * HID, HID), lambda j: (0, 0)),
            pl.BlockSpec((HID, 16), lambda j: (0, 0)),
            pl.BlockSpec((HID, 16), lambda j: (0, 0)),
        ],
        out_specs=[
            pl.BlockSpec((blk, HID), lambda j: (j, 0)),
            pl.BlockSpec((blk, 16), lambda j: (j, 0)),
            pl.BlockSpec((blk, 16), lambda j: (j, 0)),
        ],
        out_shape=[
            jax.ShapeDtypeStruct((NPAD, HID), jnp.bfloat16),
            jax.ShapeDtypeStruct((NPAD, 16), jnp.float32),
            jax.ShapeDtypeStruct((NPAD, 16), jnp.float32),
        ],
    )(h1act, w2, m2, m2s)


def _final_body(op_ref, dp_ref, b_ref, bt_ref, wfc_ref, bfc_ref, out_ref):
    d = dp_ref[0] + dp_ref[1]
    dinv = 1.0 / (d[:, 0:1] + 1e-16)
    acc = op_ref[0] + op_ref[1]
    h2act = jnp.maximum(acc * dinv + b_ref[...], 0.0)
    bt = bt_ref[...]
    gids = lax.broadcasted_iota(jnp.int32, (G, NPAD), 0)
    onehot = (bt == gids).astype(jnp.float32)
    pooled = jnp.dot(onehot, h2act, preferred_element_type=jnp.float32)
    out_ref[...] = jnp.dot(pooled, wfc_ref[...],
                           preferred_element_type=jnp.float32) + bfc_ref[...]


def _final(opart2, dpart2, b2, batch2d, wfc, bfc):
    return pl.pallas_call(
        _final_body,
        out_shape=jax.ShapeDtypeStruct((G, OUT_DIM), jnp.float32),
    )(opart2.reshape(2, NPAD, HID), dpart2.reshape(2, NPAD, 16),
      b2, batch2d, wfc, bfc)


def kernel(x, edge_index, batch, W1, att_src1, att_dst1, b1,
           W2, att_src2, att_dst2, b2, Wfc, bfc):
    f32 = jnp.float32

    xpad = jnp.pad(x, ((0, NPAD - N), (0, 0)))

    q = jnp.arange(32)
    inner = (q % 2) * 16 + q // 2
    permidx = jnp.concatenate([inner, inner + 32])

    w1r = W1.reshape(IN_DIM, HEADS1, HID).transpose(1, 0, 2)
    w1r = w1r[:, :, permidx]

    eye8 = jnp.eye(HEADS1, dtype=f32)
    as1 = att_src1[0]
    ad1 = att_dst1[0]
    m1 = jnp.concatenate(
        [as1[:, :, None] * eye8[:, None, :],
         ad1[:, :, None] * eye8[:, None, :]], axis=2)
    m1 = m1[:, permidx, :]
    m1s = jnp.concatenate([m1[:, :, 8:], m1[:, :, :8]], axis=2)

    m2 = jnp.zeros((HID, 16), f32)
    m2 = m2.at[:, 0].set(att_src2[0, 0]).at[:, 8].set(att_dst2[0, 0])
    m2 = m2[permidx, :]
    m2s = jnp.concatenate([m2[:, 8:], m2[:, :8]], axis=1)
    w2p = W2[:, permidx]

    loop = jnp.arange(N, dtype=jnp.int32)
    npadfill = jnp.full((EPAD - E - N,), N, jnp.int32)
    src = jnp.concatenate([edge_index[0], loop, npadfill])
    dst = jnp.concatenate([edge_index[1], loop, npadfill])
    src2d = src.reshape(NW, CPT * CHUNK)
    dst3d = dst.reshape(NW, CPT, CHUNK)

    rowid = jnp.arange(NPAD, dtype=jnp.int32)[:, None]
    batch2d = jnp.concatenate(
        [batch, jnp.full((NPAD - N,), G, jnp.int32)])[None, :]

    h1T, t1, t1s = _mm1(xpad, w1r, m1, m1s)
    t1 = jnp.where(rowid < N, t1, _NEG)
    t1s = jnp.where(rowid < N, t1s, _NEG)
    ex1, dpart1 = _attn_sc(t1, t1s, src, dst)
    opart1 = _msg_sc(h1T.reshape(HEADS1 * NPAD, HID), ex1, src2d, dst3d, HEADS1)
    h1act = _act1(opart1.reshape(2, HEADS1, NPAD, HID),
                  dpart1.reshape(2, NPAD, 16), b1[None, :])

    h2, t2, t2s = _mm2(h1act, w2p, m2, m2s)
    t2 = jnp.where(rowid < N, t2, _NEG)
    t2s = jnp.where(rowid < N, t2s, _NEG)
    ex2, dpart2 = _attn_sc(t2, t2s, src, dst)
    opart2 = _msg_sc(h2, ex2, src2d, dst3d, 1)

    return _final(opart2, dpart2, b2[None, :], batch2d, Wfc, bfc[None, :])


# speedup vs baseline: 1.0480x; 1.0446x over previous
import functools

import jax
import jax.numpy as jnp
from jax import lax
from jax.experimental import pallas as pl
from jax.experimental.pallas import tpu as pltpu
from jax.experimental.pallas import tpu_sc as plsc

N = 10000
E = 320000
IN_DIM = 128
HID = 64
HEADS1 = 8
OUT_DIM = 128
G = 16

NPAD = 10240
NW = 32
CHUNK = 128
IDXW = CHUNK // 128
CPT = 82
EPAD = NW * CPT * CHUNK
ROWS_PER_TILE = NPAD // 16

_NEG = -1e30


def _mm1_body(x_ref, w_ref, m_ref, ms_ref, h_ref, t_ref, ts_ref):
    h = pl.program_id(1)
    blk = jnp.dot(x_ref[...], w_ref[0], preferred_element_type=jnp.float32)
    h_ref[0] = blk.astype(jnp.bfloat16)
    t = jnp.dot(blk, m_ref[0], preferred_element_type=jnp.float32)
    ts = jnp.dot(blk, ms_ref[0], preferred_element_type=jnp.float32)

    @pl.when(h == 0)
    def _():
        t_ref[...] = t
        ts_ref[...] = ts

    @pl.when(h != 0)
    def _():
        t_ref[...] += t
        ts_ref[...] += ts


def _mm1(xpad, w1r, m1, m1s):
    nblk = NPAD // 640
    return pl.pallas_call(
        _mm1_body,
        grid=(nblk, HEADS1),
        in_specs=[
            pl.BlockSpec((640, IN_DIM), lambda j, h: (j, 0)),
            pl.BlockSpec((1, IN_DIM, HID), lambda j, h: (h, 0, 0)),
            pl.BlockSpec((1, HID, 16), lambda j, h: (h, 0, 0)),
            pl.BlockSpec((1, HID, 16), lambda j, h: (h, 0, 0)),
        ],
        out_specs=[
            pl.BlockSpec((1, 640, HID), lambda j, h: (h, j, 0)),
            pl.BlockSpec((640, 16), lambda j, h: (j, 0)),
            pl.BlockSpec((640, 16), lambda j, h: (j, 0)),
        ],
        out_shape=[
            jax.ShapeDtypeStruct((HEADS1, NPAD, HID), jnp.bfloat16),
            jax.ShapeDtypeStruct((NPAD, 16), jnp.float32),
            jax.ShapeDtypeStruct((NPAD, 16), jnp.float32),
        ],
    )(xpad, w1r, m1, m1s)


_SC_PARAMS = pltpu.CompilerParams(use_tc_tiling_on_sc=False,
                                  needs_layout_passes=False)


def _attn_sc(tmain, tswap, src, dst):
    mesh = plsc.VectorSubcoreMesh(core_axis_name="c", subcore_axis_name="s")

    @functools.partial(
        pl.kernel,
        mesh=mesh,
        compiler_params=_SC_PARAMS,
        out_type=[
            jax.ShapeDtypeStruct((16, EPAD), jnp.float32),
            jax.ShapeDtypeStruct((2 * NPAD, 16), jnp.float32),
        ],
        scratch_types=[
            pltpu.VMEM((CPT * CHUNK,), jnp.int32),
            pltpu.VMEM((CPT * CHUNK,), jnp.int32),
            pltpu.VMEM((CHUNK,), jnp.int32),
            pltpu.VMEM((CHUNK, 16), jnp.float32),
            pltpu.VMEM((CHUNK, 16), jnp.float32),
            pltpu.VMEM((CHUNK, 16), jnp.float32),
            pltpu.VMEM((16, CHUNK), jnp.float32),
            pltpu.VMEM((ROWS_PER_TILE, 16), jnp.float32),
            pltpu.VMEM_SHARED((NPAD, 16), jnp.float32),
        ],
    )
    def k(tm_hbm, tsw_hbm, src_hbm, dst_hbm, ex_hbm, dpart_hbm,
          src_all, dst_all, dst_w, ts_v, td_v, ex_v, ext_v, zbuf, shared):
        c = lax.axis_index("c")
        s = lax.axis_index("s")
        wid = s * 2 + c
        lane = lax.iota(jnp.int32, 16)
        lmask = lane < 8

        pltpu.sync_copy(src_hbm.at[wid], src_all)
        pltpu.sync_copy(dst_hbm.at[wid], dst_all)

        @pl.loop(0, ROWS_PER_TILE)
        def _(i):
            zbuf[i, :] = jnp.zeros((16,), jnp.float32)

        pltpu.sync_copy(zbuf, shared.at[pl.ds(s * ROWS_PER_TILE, ROWS_PER_TILE)])
        plsc.subcore_barrier()

        @pl.loop(0, CPT)
        def _(t):
            base = (wid * CPT + t) * CHUNK
            pltpu.sync_copy(tm_hbm.at[src_all.at[pl.ds(t * CHUNK, CHUNK)]],
                            ts_v)
            pltpu.sync_copy(tsw_hbm.at[dst_all.at[pl.ds(t * CHUNK, CHUNK)]],
                            td_v)

            @pl.loop(0, CHUNK)
            def _(i):
                su = ts_v[i, :] + td_v[i, :]
                lr = jnp.maximum(su, 0.2 * su)
                exf = jnp.exp(lr)
                exm = jnp.where(lmask, exf, 0.0)
                ex_v[i, :] = exm
                plsc.store_scatter(ext_v, [lane, jnp.full((16,), i, jnp.int32)],
                                   exm)

            for i in range(0, CHUNK, 16):
                dst_w[pl.ds(i, 16)] = dst_all[pl.ds(t * CHUNK + i, 16)]

            pltpu.sync_copy(ext_v, ex_hbm.at[:, pl.ds(base, CHUNK)])
            pltpu.sync_copy(ex_v, shared.at[dst_w], add=True)

        plsc.subcore_barrier()
        off = c * NPAD + s * ROWS_PER_TILE
        pltpu.sync_copy(shared.at[pl.ds(s * ROWS_PER_TILE, ROWS_PER_TILE)],
                        dpart_hbm.at[pl.ds(off, ROWS_PER_TILE)])

    return k(tmain, tswap, src, dst)


def _msg_sc(table, ex, src2d, dst3d, nheads):
    mesh = plsc.VectorSubcoreMesh(core_axis_name="c", subcore_axis_name="s")

    @functools.partial(
        pl.kernel,
        mesh=mesh,
        compiler_params=_SC_PARAMS,
        out_type=jax.ShapeDtypeStruct((2 * nheads * NPAD, HID), jnp.float32),
        scratch_types=[
            pltpu.VMEM((CPT * CHUNK,), jnp.int32),
            pltpu.VMEM((CPT * CHUNK,), jnp.int32),
            pltpu.VMEM((CHUNK,), jnp.int32),
            pltpu.VMEM((CHUNK,), jnp.int32),
            pltpu.VMEM((CHUNK,), jnp.int32),
            pltpu.VMEM((CHUNK,), jnp.int32),
            pltpu.VMEM((CHUNK, HID), jnp.bfloat16),
            pltpu.VMEM((CHUNK, HID), jnp.bfloat16),
            pltpu.VMEM((CHUNK, HID), jnp.float32),
            pltpu.VMEM((CHUNK, HID), jnp.float32),
            pltpu.VMEM((CHUNK,), jnp.float32),
            pltpu.VMEM((CHUNK,), jnp.float32),
            pltpu.VMEM_SHARED((NPAD, HID), jnp.float32),
            pltpu.SemaphoreType.DMA,
            pltpu.SemaphoreType.DMA,
            pltpu.SemaphoreType.DMA,
            pltpu.SemaphoreType.DMA,
            pltpu.SemaphoreType.DMA,
            pltpu.SemaphoreType.DMA,
        ],
    )
    def k(tab_hbm, ex_hbm, src_hbm, dst_hbm, opart_hbm,
          src_all, dst_all, dw0, dw1, idx0, idx1, ri0, ri1, ro0, ro1,
          exb0, exb1, shared,
          sg0, sg1, se0, se1, ss0, ss1):
        c = lax.axis_index("c")
        s = lax.axis_index("s")
        wid = s * 2 + c
        dw = (dw0, dw1)
        idx = (idx0, idx1)
        ri = (ri0, ri1)
        ro = (ro0, ro1)
        exb = (exb0, exb1)
        sg = (sg0, sg1)
        se = (se0, se1)
        ss = (ss0, ss1)

        pltpu.sync_copy(src_hbm.at[wid], src_all)
        pltpu.sync_copy(dst_hbm.at[wid], dst_all)

        def issue(t, b, h):
            if h == 0:
                gidx = src_all.at[pl.ds(t * CHUNK, CHUNK)]
            else:
                for i in range(0, CHUNK, 16):
                    idx[b][pl.ds(i, 16)] = (
                        src_all[pl.ds(t * CHUNK + i, 16)] + h * NPAD)
                gidx = idx[b]
            pltpu.async_copy(tab_hbm.at[gidx], ri[b], sg[b])
            ebase = (wid * CPT + t) * CHUNK
            pltpu.async_copy(ex_hbm.at[h, pl.ds(ebase, CHUNK)], exb[b], se[b])

        def step(t, b, h):
            pltpu.make_async_copy(tab_hbm.at[idx[b]], ri[b], sg[b]).wait()
            pltpu.make_async_copy(
                ex_hbm.at[0, pl.ds(0, CHUNK)], exb[b], se[b]).wait()

            @pl.when(t >= 2)
            def _():
                pltpu.make_async_copy(
                    ro[b], shared.at[dw[b]], ss[b]).wait()

            @pl.loop(0, CHUNK, step=16)
            def _(i0):
                av = exb[b][pl.ds(i0, 16)]
                for u in range(16):
                    i = i0 + u
                    a = av[u]
                    for j in range(HID // 32):
                        lo, hi = plsc.unpack(
                            ri[b][i, pl.ds(j * 32, 32)],
                            format=plsc.PackFormat.INTERLEAVED)
                        ro[b][i, pl.ds(j * 32, 16)] = lo * a
                        ro[b][i, pl.ds(j * 32 + 16, 16)] = hi * a

            for i in range(0, CHUNK, 16):
                dw[b][pl.ds(i, 16)] = dst_all[pl.ds(t * CHUNK + i, 16)]

            pltpu.async_copy(ro[b], shared.at[dw[b]], ss[b], add=True)

            @pl.when(t + 2 < CPT)
            def _():
                issue(t + 2, b, h)

        for h in range(nheads):
            @pl.loop(0, CHUNK)
            def _(i):
                for j in range(HID // 16):
                    ro0[i, pl.ds(j * 16, 16)] = jnp.zeros((16,), jnp.float32)

            for kk in range(ROWS_PER_TILE // 128):
                pltpu.sync_copy(
                    ro0.at[pl.ds(0, 128)],
                    shared.at[pl.ds(s * ROWS_PER_TILE + kk * 128, 128)])
            plsc.subcore_barrier()

            issue(0, 0, h)
            issue(1, 1, h)

            @pl.loop(0, CPT, step=2)
            def _(t):
                step(t, 0, h)
                step(t + 1, 1, h)

            for b in range(2):
                pltpu.make_async_copy(
                    ro[b], shared.at[dw[b]], ss[b]).wait()

            plsc.subcore_barrier()
            off = (c * nheads + h) * NPAD + s * ROWS_PER_TILE
            pltpu.sync_copy(shared.at[pl.ds(s * ROWS_PER_TILE, ROWS_PER_TILE)],
                            opart_hbm.at[pl.ds(off, ROWS_PER_TILE)])

    return k(table, ex, src2d, dst3d)


def _act1_body(op_ref, dp_ref, b_ref, out_ref):
    dsum = dp_ref[0] + dp_ref[1]
    dinv = 1.0 / (dsum + 1e-16)
    acc = op_ref[0] + op_ref[1]
    parts = [acc[h] * dinv[:, h:h + 1] for h in range(HEADS1)]
    cat = jnp.concatenate(parts, axis=1)
    out_ref[...] = jnp.maximum(cat + b_ref[...], 0.0)


def _act1(opart, dpart, b1):
    blk = 1280
    nblk = NPAD // blk
    return pl.pallas_call(
        _act1_body,
        grid=(nblk,),
        in_specs=[
            pl.BlockSpec((2, HEADS1, blk, HID), lambda j: (0, 0, j, 0)),
            pl.BlockSpec((2, blk, 16), lambda j: (0, j, 0)),
            pl.BlockSpec((1, HEADS1 * HID), lambda j: (0, 0)),
        ],
        out_specs=pl.BlockSpec((blk, HEADS1 * HID), lambda j: (j, 0)),
        out_shape=jax.ShapeDtypeStruct((NPAD, HEADS1 * HID), jnp.float32),
    )(opart, dpart, b1)


def _mm2_body(x_ref, w_ref, m_ref, ms_ref, h_ref, t_ref, ts_ref):
    blk = jnp.dot(x_ref[...], w_ref[...], preferred_element_type=jnp.float32)
    h_ref[...] = blk.astype(jnp.bfloat16)
    t_ref[...] = jnp.dot(blk, m_ref[...], preferred_element_type=jnp.float32)
    ts_ref[...] = jnp.dot(blk, ms_ref[...], preferred_element_type=jnp.float32)


def _mm2(h1act, w2, m2, m2s):
    blk = 1280
    nblk = NPAD // blk
    return pl.pallas_call(
        _mm2_body,
        grid=(nblk,),
        in_specs=[
            pl.BlockSpec((blk, HEADS1 * HID), lambda j: (j, 0)),
            pl.BlockSpec((HEADS1 * HID, HID), lambda j: (0, 0)),
            pl.BlockSpec((HID, 16), lambda j: (0, 0)),
            pl.BlockSpec((HID, 16), lambda j: (0, 0)),
        ],
        out_specs=[
            pl.BlockSpec((blk, HID), lambda j: (j, 0)),
            pl.BlockSpec((blk, 16), lambda j: (j, 0)),
            pl.BlockSpec((blk, 16), lambda j: (j, 0)),
        ],
        out_shape=[
            jax.ShapeDtypeStruct((NPAD, HID), jnp.bfloat16),
            jax.ShapeDtypeStruct((NPAD, 16), jnp.float32),
            jax.ShapeDtypeStruct((NPAD, 16), jnp.float32),
        ],
    )(h1act, w2, m2, m2s)


def _final_body(op_ref, dp_ref, b_ref, bt_ref, wfc_ref, bfc_ref, out_ref):
    d = dp_ref[0] + dp_ref[1]
    dinv = 1.0 / (d[:, 0:1] + 1e-16)
    acc = op_ref[0] + op_ref[1]
    h2act = jnp.maximum(acc * dinv + b_ref[...], 0.0)
    bt = bt_ref[...]
    gids = lax.broadcasted_iota(jnp.int32, (G, NPAD), 0)
    onehot = (bt == gids).astype(jnp.float32)
    pooled = jnp.dot(onehot, h2act, preferred_element_type=jnp.float32)
    out_ref[...] = jnp.dot(pooled, wfc_ref[...],
                           preferred_element_type=jnp.float32) + bfc_ref[...]


def _final(opart2, dpart2, b2, batch2d, wfc, bfc):
    return pl.pallas_call(
        _final_body,
        out_shape=jax.ShapeDtypeStruct((G, OUT_DIM), jnp.float32),
    )(opart2.reshape(2, NPAD, HID), dpart2.reshape(2, NPAD, 16),
      b2, batch2d, wfc, bfc)


def kernel(x, edge_index, batch, W1, att_src1, att_dst1, b1,
           W2, att_src2, att_dst2, b2, Wfc, bfc):
    f32 = jnp.float32

    xpad = jnp.pad(x, ((0, NPAD - N), (0, 0)))

    q = jnp.arange(32)
    inner = (q % 2) * 16 + q // 2
    permidx = jnp.concatenate([inner, inner + 32])

    w1r = W1.reshape(IN_DIM, HEADS1, HID).transpose(1, 0, 2)
    w1r = w1r[:, :, permidx]

    eye8 = jnp.eye(HEADS1, dtype=f32)
    as1 = att_src1[0]
    ad1 = att_dst1[0]
    m1 = jnp.concatenate(
        [as1[:, :, None] * eye8[:, None, :],
         ad1[:, :, None] * eye8[:, None, :]], axis=2)
    m1 = m1[:, permidx, :]
    m1s = jnp.concatenate([m1[:, :, 8:], m1[:, :, :8]], axis=2)

    m2 = jnp.zeros((HID, 16), f32)
    m2 = m2.at[:, 0].set(att_src2[0, 0]).at[:, 8].set(att_dst2[0, 0])
    m2 = m2[permidx, :]
    m2s = jnp.concatenate([m2[:, 8:], m2[:, :8]], axis=1)
    w2p = W2[:, permidx]

    loop = jnp.arange(N, dtype=jnp.int32)
    npadfill = jnp.full((EPAD - E - N,), N, jnp.int32)
    src = jnp.concatenate([edge_index[0], loop, npadfill])
    dst = jnp.concatenate([edge_index[1], loop, npadfill])
    src2 = src.reshape(NW, CPT * CHUNK)
    dst2 = dst.reshape(NW, CPT * CHUNK)

    rowid = jnp.arange(NPAD, dtype=jnp.int32)[:, None]
    batch2d = jnp.concatenate(
        [batch, jnp.full((NPAD - N,), G, jnp.int32)])[None, :]

    h1T, t1, t1s = _mm1(xpad, w1r, m1, m1s)
    t1 = jnp.where(rowid < N, t1, _NEG)
    t1s = jnp.where(rowid < N, t1s, _NEG)
    ex1, dpart1 = _attn_sc(t1, t1s, src2, dst2)
    opart1 = _msg_sc(h1T.reshape(HEADS1 * NPAD, HID), ex1, src2, dst2, HEADS1)
    h1act = _act1(opart1.reshape(2, HEADS1, NPAD, HID),
                  dpart1.reshape(2, NPAD, 16), b1[None, :])

    h2, t2, t2s = _mm2(h1act, w2p, m2, m2s)
    t2 = jnp.where(rowid < N, t2, _NEG)
    t2s = jnp.where(rowid < N, t2s, _NEG)
    ex2, dpart2 = _attn_sc(t2, t2s, src2, dst2)
    opart2 = _msg_sc(h2, ex2, src2, dst2, 1)

    return _final(opart2, dpart2, b2[None, :], batch2d, Wfc, bfc[None, :])


# speedup vs baseline: 1.0851x; 1.0355x over previous
import functools

import jax
import jax.numpy as jnp
from jax import lax
from jax.experimental import pallas as pl
from jax.experimental.pallas import tpu as pltpu
from jax.experimental.pallas import tpu_sc as plsc

N = 10000
E = 320000
IN_DIM = 128
HID = 64
HEADS1 = 8
OUT_DIM = 128
G = 16

NPAD = 10240
NW = 32
CHUNK = 128
CPT = 84
EPAD = NW * CPT * CHUNK
ROWS_PER_TILE = NPAD // 16

_NEG = -1e30


def _mm1_body(x_ref, w_ref, m_ref, ms_ref, h_ref, t_ref, ts_ref):
    h = pl.program_id(1)
    blk = jnp.dot(x_ref[...], w_ref[0], preferred_element_type=jnp.float32)
    h_ref[0] = blk.astype(jnp.bfloat16)
    t = jnp.dot(blk, m_ref[0], preferred_element_type=jnp.float32)
    ts = jnp.dot(blk, ms_ref[0], preferred_element_type=jnp.float32)

    @pl.when(h == 0)
    def _():
        t_ref[...] = t
        ts_ref[...] = ts

    @pl.when(h != 0)
    def _():
        t_ref[...] += t
        ts_ref[...] += ts


def _mm1(xpad, w1r, m1, m1s):
    nblk = NPAD // 640
    return pl.pallas_call(
        _mm1_body,
        grid=(nblk, HEADS1),
        in_specs=[
            pl.BlockSpec((640, IN_DIM), lambda j, h: (j, 0)),
            pl.BlockSpec((1, IN_DIM, HID), lambda j, h: (h, 0, 0)),
            pl.BlockSpec((1, HID, 16), lambda j, h: (h, 0, 0)),
            pl.BlockSpec((1, HID, 16), lambda j, h: (h, 0, 0)),
        ],
        out_specs=[
            pl.BlockSpec((1, 640, HID), lambda j, h: (h, j, 0)),
            pl.BlockSpec((640, 16), lambda j, h: (j, 0)),
            pl.BlockSpec((640, 16), lambda j, h: (j, 0)),
        ],
        out_shape=[
            jax.ShapeDtypeStruct((HEADS1, NPAD, HID), jnp.bfloat16),
            jax.ShapeDtypeStruct((NPAD, 16), jnp.float32),
            jax.ShapeDtypeStruct((NPAD, 16), jnp.float32),
        ],
    )(xpad, w1r, m1, m1s)


_SC_PARAMS = pltpu.CompilerParams(use_tc_tiling_on_sc=False,
                                  needs_layout_passes=False)


def _attn_sc(tmain, tswap, src, dst):
    mesh = plsc.VectorSubcoreMesh(core_axis_name="c", subcore_axis_name="s")

    @functools.partial(
        pl.kernel,
        mesh=mesh,
        compiler_params=_SC_PARAMS,
        out_type=[
            jax.ShapeDtypeStruct((16, EPAD), jnp.float32),
            jax.ShapeDtypeStruct((2 * NPAD, 16), jnp.float32),
        ],
        scratch_types=[
            pltpu.VMEM((CPT * CHUNK,), jnp.int32),
            pltpu.VMEM((CPT * CHUNK,), jnp.int32),
            pltpu.VMEM((CHUNK,), jnp.int32),
            pltpu.VMEM((CHUNK,), jnp.int32),
            pltpu.VMEM((CHUNK, 16), jnp.float32),
            pltpu.VMEM((CHUNK, 16), jnp.float32),
            pltpu.VMEM((CHUNK, 16), jnp.float32),
            pltpu.VMEM((CHUNK, 16), jnp.float32),
            pltpu.VMEM((CHUNK, 16), jnp.float32),
            pltpu.VMEM((CHUNK, 16), jnp.float32),
            pltpu.VMEM((16, CHUNK), jnp.float32),
            pltpu.VMEM((16, CHUNK), jnp.float32),
            pltpu.VMEM((ROWS_PER_TILE, 16), jnp.float32),
            pltpu.VMEM_SHARED((NPAD, 16), jnp.float32),
            pltpu.SemaphoreType.DMA,
            pltpu.SemaphoreType.DMA,
            pltpu.SemaphoreType.DMA,
            pltpu.SemaphoreType.DMA,
            pltpu.SemaphoreType.DMA,
            pltpu.SemaphoreType.DMA,
            pltpu.SemaphoreType.DMA,
            pltpu.SemaphoreType.DMA,
        ],
    )
    def k(tm_hbm, tsw_hbm, src_hbm, dst_hbm, ex_hbm, dpart_hbm,
          src_all, dst_all, dw0, dw1, ts0, ts1, td0, td1, ex0, ex1,
          ext0, ext1, zbuf, shared,
          sgs0, sgs1, sgd0, sgd1, sx0, sx1, sd0, sd1):
        c = lax.axis_index("c")
        s = lax.axis_index("s")
        wid = s * 2 + c
        lane = lax.iota(jnp.int32, 16)
        lmask = lane < 8
        dw = (dw0, dw1)
        ts = (ts0, ts1)
        td = (td0, td1)
        exv = (ex0, ex1)
        ext = (ext0, ext1)
        sgs = (sgs0, sgs1)
        sgd = (sgd0, sgd1)
        sx = (sx0, sx1)
        sd = (sd0, sd1)

        pltpu.sync_copy(src_hbm.at[wid], src_all)
        pltpu.sync_copy(dst_hbm.at[wid], dst_all)

        @pl.loop(0, ROWS_PER_TILE)
        def _(i):
            zbuf[i, :] = jnp.zeros((16,), jnp.float32)

        pltpu.sync_copy(zbuf, shared.at[pl.ds(s * ROWS_PER_TILE, ROWS_PER_TILE)])
        plsc.subcore_barrier()

        def issue(t, b):
            sl = pl.ds(t * CHUNK, CHUNK)
            pltpu.async_copy(tm_hbm.at[src_all.at[sl]], ts[b], sgs[b])
            pltpu.async_copy(tsw_hbm.at[dst_all.at[sl]], td[b], sgd[b])

        def step(t, b):
            base = (wid * CPT + t) * CHUNK
            pltpu.make_async_copy(tm_hbm.at[dw[b]], ts[b], sgs[b]).wait()
            pltpu.make_async_copy(tsw_hbm.at[dw[b]], td[b], sgd[b]).wait()

            @pl.when(t >= 2)
            def _():
                pltpu.make_async_copy(
                    ext[b], ex_hbm.at[:, pl.ds(0, CHUNK)], sx[b]).wait()
                pltpu.make_async_copy(
                    exv[b], shared.at[dw[b]], sd[b]).wait()

            @pl.loop(0, CHUNK)
            def _(i):
                su = ts[b][i, :] + td[b][i, :]
                lr = jnp.maximum(su, 0.2 * su)
                exf = jnp.exp(lr)
                exm = jnp.where(lmask, exf, 0.0)
                exv[b][i, :] = exm
                plsc.store_scatter(
                    ext[b], [lane, jnp.full((16,), i, jnp.int32)], exm)

            for i in range(0, CHUNK, 16):
                dw[b][pl.ds(i, 16)] = dst_all[pl.ds(t * CHUNK + i, 16)]

            pltpu.async_copy(ext[b], ex_hbm.at[:, pl.ds(base, CHUNK)], sx[b])
            pltpu.async_copy(exv[b], shared.at[dw[b]], sd[b], add=True)

            @pl.when(t + 2 < CPT)
            def _():
                issue(t + 2, b)

        issue(0, 0)
        issue(1, 1)

        @pl.loop(0, CPT, step=2)
        def _(t):
            step(t, 0)
            step(t + 1, 1)

        for b in range(2):
            pltpu.make_async_copy(
                ext[b], ex_hbm.at[:, pl.ds(0, CHUNK)], sx[b]).wait()
            pltpu.make_async_copy(exv[b], shared.at[dw[b]], sd[b]).wait()

        plsc.subcore_barrier()
        off = c * NPAD + s * ROWS_PER_TILE
        pltpu.sync_copy(shared.at[pl.ds(s * ROWS_PER_TILE, ROWS_PER_TILE)],
                        dpart_hbm.at[pl.ds(off, ROWS_PER_TILE)])

    return k(tmain, tswap, src, dst)


def _msg_sc(table, ex, src2d, dst3d, nheads):
    mesh = plsc.VectorSubcoreMesh(core_axis_name="c", subcore_axis_name="s")

    @functools.partial(
        pl.kernel,
        mesh=mesh,
        compiler_params=_SC_PARAMS,
        out_type=jax.ShapeDtypeStruct((2 * nheads * NPAD, HID), jnp.float32),
        scratch_types=(
            [
                pltpu.VMEM((CPT * CHUNK,), jnp.int32),
                pltpu.VMEM((CPT * CHUNK,), jnp.int32),
            ]
            + [pltpu.VMEM((CHUNK,), jnp.int32)] * 4
            + [pltpu.VMEM((CHUNK,), jnp.int32)] * 4
            + [pltpu.VMEM((CHUNK, HID), jnp.bfloat16)] * 4
            + [pltpu.VMEM((CHUNK, HID), jnp.float32)] * 4
            + [pltpu.VMEM((CHUNK,), jnp.float32)] * 4
            + [pltpu.VMEM_SHARED((NPAD, HID), jnp.float32)]
            + [pltpu.SemaphoreType.DMA] * 12
        ),
    )
    def k(tab_hbm, ex_hbm, src_hbm, dst_hbm, opart_hbm,
          src_all, dst_all,
          dw0, dw1, dw2, dw3, idx0, idx1, idx2, idx3,
          ri0, ri1, ri2, ri3, ro0, ro1, ro2, ro3,
          exb0, exb1, exb2, exb3, shared,
          sg0, sg1, sg2, sg3, se0, se1, se2, se3, ss0, ss1, ss2, ss3):
        c = lax.axis_index("c")
        s = lax.axis_index("s")
        wid = s * 2 + c
        dw = (dw0, dw1, dw2, dw3)
        idx = (idx0, idx1, idx2, idx3)
        ri = (ri0, ri1, ri2, ri3)
        ro = (ro0, ro1, ro2, ro3)
        exb = (exb0, exb1, exb2, exb3)
        sg = (sg0, sg1, sg2, sg3)
        se = (se0, se1, se2, se3)
        ss = (ss0, ss1, ss2, ss3)
        nbuf = 4

        pltpu.sync_copy(src_hbm.at[wid], src_all)
        pltpu.sync_copy(dst_hbm.at[wid], dst_all)

        def issue(t, b, h):
            for i in range(0, CHUNK, 16):
                idx[b][pl.ds(i, 16)] = (
                    src_all[pl.ds(t * CHUNK + i, 16)] + h * NPAD)
            pltpu.async_copy(tab_hbm.at[idx[b]], ri[b], sg[b])
            ebase = (wid * CPT + t) * CHUNK
            pltpu.async_copy(ex_hbm.at[h, pl.ds(ebase, CHUNK)], exb[b], se[b])

        def step(t, b, h):
            pltpu.make_async_copy(tab_hbm.at[idx[b]], ri[b], sg[b]).wait()
            pltpu.make_async_copy(
                ex_hbm.at[0, pl.ds(0, CHUNK)], exb[b], se[b]).wait()

            @pl.when(t >= nbuf)
            def _():
                pltpu.make_async_copy(
                    ro[b], shared.at[dw[b]], ss[b]).wait()

            @pl.loop(0, CHUNK, step=16)
            def _(i0):
                av = exb[b][pl.ds(i0, 16)]
                for u in range(16):
                    i = i0 + u
                    a = av[u]
                    for j in range(HID // 32):
                        lo, hi = plsc.unpack(
                            ri[b][i, pl.ds(j * 32, 32)],
                            format=plsc.PackFormat.INTERLEAVED)
                        ro[b][i, pl.ds(j * 32, 16)] = lo * a
                        ro[b][i, pl.ds(j * 32 + 16, 16)] = hi * a

            for i in range(0, CHUNK, 16):
                dw[b][pl.ds(i, 16)] = dst_all[pl.ds(t * CHUNK + i, 16)]

            pltpu.async_copy(ro[b], shared.at[dw[b]], ss[b], add=True)

            @pl.when(t + nbuf < CPT)
            def _():
                issue(t + nbuf, b, h)

        @pl.loop(0, nheads)
        def _(h):
            @pl.loop(0, CHUNK)
            def _(i):
                for j in range(HID // 16):
                    ro0[i, pl.ds(j * 16, 16)] = jnp.zeros((16,), jnp.float32)

            for kk in range(ROWS_PER_TILE // 128):
                pltpu.sync_copy(
                    ro0.at[pl.ds(0, 128)],
                    shared.at[pl.ds(s * ROWS_PER_TILE + kk * 128, 128)])
            plsc.subcore_barrier()

            for b in range(nbuf):
                issue(b, b, h)

            @pl.loop(0, CPT, step=nbuf)
            def _(t):
                for b in range(nbuf):
                    step(t + b, b, h)

            for b in range(nbuf):
                pltpu.make_async_copy(
                    ro[b], shared.at[dw[b]], ss[b]).wait()

            plsc.subcore_barrier()
            off = (c * nheads + h) * NPAD + s * ROWS_PER_TILE
            pltpu.sync_copy(shared.at[pl.ds(s * ROWS_PER_TILE, ROWS_PER_TILE)],
                            opart_hbm.at[pl.ds(off, ROWS_PER_TILE)])

    return k(table, ex, src2d, dst3d)


def _act1_body(op_ref, dp_ref, b_ref, out_ref):
    dsum = dp_ref[0] + dp_ref[1]
    dinv = 1.0 / (dsum + 1e-16)
    acc = op_ref[0] + op_ref[1]
    parts = [acc[h] * dinv[:, h:h + 1] for h in range(HEADS1)]
    cat = jnp.concatenate(parts, axis=1)
    out_ref[...] = jnp.maximum(cat + b_ref[...], 0.0)


def _act1(opart, dpart, b1):
    blk = 1280
    nblk = NPAD // blk
    return pl.pallas_call(
        _act1_body,
        grid=(nblk,),
        in_specs=[
            pl.BlockSpec((2, HEADS1, blk, HID), lambda j: (0, 0, j, 0)),
            pl.BlockSpec((2, blk, 16), lambda j: (0, j, 0)),
            pl.BlockSpec((1, HEADS1 * HID), lambda j: (0, 0)),
        ],
        out_specs=pl.BlockSpec((blk, HEADS1 * HID), lambda j: (j, 0)),
        out_shape=jax.ShapeDtypeStruct((NPAD, HEADS1 * HID), jnp.float32),
    )(opart, dpart, b1)


def _mm2_body(x_ref, w_ref, m_ref, ms_ref, h_ref, t_ref, ts_ref):
    blk = jnp.dot(x_ref[...], w_ref[...], preferred_element_type=jnp.float32)
    h_ref[...] = blk.astype(jnp.bfloat16)
    t_ref[...] = jnp.dot(blk, m_ref[...], preferred_element_type=jnp.float32)
    ts_ref[...] = jnp.dot(blk, ms_ref[...], preferred_element_type=jnp.float32)


def _mm2(h1act, w2, m2, m2s):
    blk = 1280
    nblk = NPAD // blk
    return pl.pallas_call(
        _mm2_body,
        grid=(nblk,),
        in_specs=[
            pl.BlockSpec((blk, HEADS1 * HID), lambda j: (j, 0)),
            pl.BlockSpec((HEADS1 * HID, HID), lambda j: (0, 0)),
            pl.BlockSpec((HID, 16), lambda j: (0, 0)),
            pl.BlockSpec((HID, 16), lambda j: (0, 0)),
        ],
        out_specs=[
            pl.BlockSpec((blk, HID), lambda j: (j, 0)),
            pl.BlockSpec((blk, 16), lambda j: (j, 0)),
            pl.BlockSpec((blk, 16), lambda j: (j, 0)),
        ],
        out_shape=[
            jax.ShapeDtypeStruct((NPAD, HID), jnp.bfloat16),
            jax.ShapeDtypeStruct((NPAD, 16), jnp.float32),
            jax.ShapeDtypeStruct((NPAD, 16), jnp.float32),
        ],
    )(h1act, w2, m2, m2s)


def _final_body(op_ref, dp_ref, b_ref, bt_ref, wfc_ref, bfc_ref, out_ref):
    d = dp_ref[0] + dp_ref[1]
    dinv = 1.0 / (d[:, 0:1] + 1e-16)
    acc = op_ref[0] + op_ref[1]
    h2act = jnp.maximum(acc * dinv + b_ref[...], 0.0)
    bt = bt_ref[...]
    gids = lax.broadcasted_iota(jnp.int32, (G, NPAD), 0)
    onehot = (bt == gids).astype(jnp.float32)
    pooled = jnp.dot(onehot, h2act, preferred_element_type=jnp.float32)
    out_ref[...] = jnp.dot(pooled, wfc_ref[...],
                           preferred_element_type=jnp.float32) + bfc_ref[...]


def _final(opart2, dpart2, b2, batch2d, wfc, bfc):
    return pl.pallas_call(
        _final_body,
        out_shape=jax.ShapeDtypeStruct((G, OUT_DIM), jnp.float32),
    )(opart2.reshape(2, NPAD, HID), dpart2.reshape(2, NPAD, 16),
      b2, batch2d, wfc, bfc)


def kernel(x, edge_index, batch, W1, att_src1, att_dst1, b1,
           W2, att_src2, att_dst2, b2, Wfc, bfc):
    f32 = jnp.float32

    xpad = jnp.pad(x, ((0, NPAD - N), (0, 0)))

    q = jnp.arange(32)
    inner = (q % 2) * 16 + q // 2
    permidx = jnp.concatenate([inner, inner + 32])

    w1r = W1.reshape(IN_DIM, HEADS1, HID).transpose(1, 0, 2)
    w1r = w1r[:, :, permidx]

    eye8 = jnp.eye(HEADS1, dtype=f32)
    as1 = att_src1[0]
    ad1 = att_dst1[0]
    m1 = jnp.concatenate(
        [as1[:, :, None] * eye8[:, None, :],
         ad1[:, :, None] * eye8[:, None, :]], axis=2)
    m1 = m1[:, permidx, :]
    m1s = jnp.concatenate([m1[:, :, 8:], m1[:, :, :8]], axis=2)

    m2 = jnp.zeros((HID, 16), f32)
    m2 = m2.at[:, 0].set(att_src2[0, 0]).at[:, 8].set(att_dst2[0, 0])
    m2 = m2[permidx, :]
    m2s = jnp.concatenate([m2[:, 8:], m2[:, :8]], axis=1)
    w2p = W2[:, permidx]

    loop = jnp.arange(N, dtype=jnp.int32)
    npadfill = jnp.full((EPAD - E - N,), N, jnp.int32)
    src = jnp.concatenate([edge_index[0], loop, npadfill])
    dst = jnp.concatenate([edge_index[1], loop, npadfill])
    src2 = src.reshape(NW, CPT * CHUNK)
    dst2 = dst.reshape(NW, CPT * CHUNK)

    rowid = jnp.arange(NPAD, dtype=jnp.int32)[:, None]
    batch2d = jnp.concatenate(
        [batch, jnp.full((NPAD - N,), G, jnp.int32)])[None, :]

    h1T, t1, t1s = _mm1(xpad, w1r, m1, m1s)
    t1 = jnp.where(rowid < N, t1, _NEG)
    t1s = jnp.where(rowid < N, t1s, _NEG)
    ex1, dpart1 = _attn_sc(t1, t1s, src2, dst2)
    opart1 = _msg_sc(h1T.reshape(HEADS1 * NPAD, HID), ex1, src2, dst2, HEADS1)
    h1act = _act1(opart1.reshape(2, HEADS1, NPAD, HID),
                  dpart1.reshape(2, NPAD, 16), b1[None, :])

    h2, t2, t2s = _mm2(h1act, w2p, m2, m2s)
    t2 = jnp.where(rowid < N, t2, _NEG)
    t2s = jnp.where(rowid < N, t2s, _NEG)
    ex2, dpart2 = _attn_sc(t2, t2s, src2, dst2)
    opart2 = _msg_sc(h2, ex2, src2, dst2, 1)

    return _final(opart2, dpart2, b2[None, :], batch2d, Wfc, bfc[None, :])
